# Initial kernel scaffold; baseline (speedup 1.0000x reference)
import jax
import jax.numpy as jnp
from jax.experimental import pallas as pl


def kernel(feat, gamma, beta, modality_id):
    raise NotImplementedError("write your pallas kernel here")


# speedup vs baseline: 4.3292x; 4.3292x over previous
import jax
import jax.numpy as jnp
from jax.experimental import pallas as pl
from jax.experimental.pallas import tpu as pltpu

_ROW_BLOCK = 512


def _body(mid_ref, feat_ref, g_ref, b_ref, out_ref):
    out_ref[...] = feat_ref[...] * g_ref[0] + b_ref[0]


def kernel(feat, gamma, beta, modality_id):
    n, d = feat.shape
    nm = gamma.shape[0]
    mid = jnp.clip(jnp.atleast_1d(jnp.asarray(modality_id, dtype=jnp.int32)), 0, nm - 1)
    gamma3 = gamma.reshape(nm, 1, d)
    beta3 = beta.reshape(nm, 1, d)
    return pl.pallas_call(
        _body,
        grid_spec=pltpu.PrefetchScalarGridSpec(
            num_scalar_prefetch=1,
            grid=(n // _ROW_BLOCK,),
            in_specs=[
                pl.BlockSpec((_ROW_BLOCK, d), lambda i, m: (i, 0)),
                pl.BlockSpec((1, 1, d), lambda i, m: (m[0], 0, 0)),
                pl.BlockSpec((1, 1, d), lambda i, m: (m[0], 0, 0)),
            ],
            out_specs=pl.BlockSpec((_ROW_BLOCK, d), lambda i, m: (i, 0)),
        ),
        out_shape=jax.ShapeDtypeStruct((n, d), feat.dtype),
    )(mid, feat, gamma3, beta3)
